# baseline (speedup 1.0000x reference)
import dataclasses

import jax
import jax.numpy as jnp
from jax import lax
from jax.experimental import pallas as pl
from jax.experimental.pallas import tpu as pltpu
from jax.experimental.pallas import tpu_sc as plsc

N_NODES = 10000
D = 128
N_EDGES = 320000

NC = 2
NS = 16
NW = NC * NS
CH = 128
NCHUNK = N_EDGES // CH
NCHUNKP = 2560
NPAD = NCHUNKP * CH - N_EDGES

SPAN = 624
TAIL_BASE = NS * SPAN
TAIL = N_NODES - TAIL_BASE

NCPT_DEG = NCHUNKP // NW

NCPT0 = 96
NCPT1 = 64
CORE1_BASE = NS * NCPT0
GRP = 16
NGRP0 = NCPT0 // GRP
NGRP1 = NCPT1 // GRP
SLOTS = 2
GITER = GRP // SLOTS

_mesh = plsc.VectorSubcoreMesh(core_axis_name="c", subcore_axis_name="s")

_sc_params = pltpu.CompilerParams()
if "needs_layout_passes" in pltpu.CompilerParams.__dataclass_fields__:
    _sc_params = dataclasses.replace(_sc_params, needs_layout_passes=False)


def _span_copy(sid, src, dst):
    base = sid * SPAN
    pltpu.sync_copy(src.at[pl.ds(base, SPAN)], dst.at[pl.ds(base, SPAN)])

    @pl.when(sid == NS - 1)
    def _():
        pltpu.sync_copy(src.at[pl.ds(TAIL_BASE, TAIL)],
                        dst.at[pl.ds(TAIL_BASE, TAIL)])


def _deg_hist_body(ei_hbm, out_hbm, idx_v, deg_v):
    cid = lax.axis_index("c")
    sid = lax.axis_index("s")
    wid = sid * NC + cid
    start = wid * NCPT_DEG

    @pl.loop(0, N_NODES // 16)
    def _(r):
        deg_v[pl.ds(r * 16, 16)] = jnp.zeros((16,), jnp.float32)

    pltpu.sync_copy(ei_hbm.at[1, pl.ds(start, NCPT_DEG)], idx_v)

    ones = jnp.ones((16,), jnp.float32)
    nloc = jnp.clip(NCHUNK - start, 0, NCPT_DEG)

    @pl.loop(0, nloc)
    def _(c):
        for j in range(CH // 16):
            idx16 = idx_v[c, pl.ds(j * 16, 16)]
            plsc.addupdate_scatter(deg_v, [idx16], ones)

    pltpu.sync_copy(deg_v, out_hbm.at[pl.ds(wid * N_NODES, N_NODES)])


def _agg_body(y_hbm, ei_hbm, zeros_hbm, out_hbm,
              rowi_v, coli_v, buf0, buf1, g0, g1, s0, s1, acc_sh):
    cid = lax.axis_index("c")
    sid = lax.axis_index("s")
    bufs = (buf0, buf1)
    gsems = (g0, g1)
    ssems = (s0, s1)

    _span_copy(sid, zeros_hbm, acc_sh)
    plsc.subcore_barrier()

    def load_idx(gs):
        pltpu.sync_copy(ei_hbm.at[0, pl.ds(gs, GRP)], rowi_v)
        pltpu.sync_copy(ei_hbm.at[1, pl.ds(gs, GRP)], coli_v)

    def g_start(b, j):
        pltpu.make_async_copy(y_hbm.at[rowi_v.at[j]], bufs[b],
                              gsems[b]).start()

    def g_wait(b):
        pltpu.make_async_copy(y_hbm.at[rowi_v.at[0]], bufs[b],
                              gsems[b]).wait()

    def s_start(b, j):
        pltpu.make_async_copy(bufs[b], acc_sh.at[coli_v.at[j]],
                              ssems[b]).start(add=True)

    def s_wait(b):
        pltpu.make_async_copy(bufs[b], acc_sh.at[coli_v.at[0]],
                              ssems[b]).wait()

    @pl.when(cid == 0)
    def _core0_async():
        start = sid * NCPT0

        @pl.loop(0, NGRP0)
        def _(g):
            load_idx(start + g * GRP)
            for b in range(SLOTS):
                g_start(b, b)

            @pl.loop(0, GITER)
            def _(i):
                base = i * SLOTS
                for b in range(SLOTS):
                    g_wait(b)
                    s_start(b, base + b)
                for b in range(SLOTS):
                    s_wait(b)
                    nxt = base + SLOTS + b

                    @pl.when(nxt < GRP)
                    def _():
                        g_start(b, nxt)

    @pl.when(cid == 1)
    def _core1_sync():
        start = CORE1_BASE + sid * NCPT1

        @pl.loop(0, NGRP1)
        def _(g):
            load_idx(start + g * GRP)

            @pl.loop(0, GRP)
            def _(j):
                pltpu.sync_copy(y_hbm.at[rowi_v.at[j]], buf0)
                pltpu.sync_copy(buf0, acc_sh.at[coli_v.at[j]], add=True)

    plsc.subcore_barrier()
    _span_copy(sid, acc_sh, out_hbm.at[cid])


def _dis_body(parts_ref, dis_ref):
    deg = jnp.sum(parts_ref[...], axis=0, keepdims=True) + 1.0
    dis_ref[...] = jnp.transpose(lax.rsqrt(deg), (1, 0))


def _linear_body(x_ref, w_ref, dis_ref, y_ref):
    y_ref[...] = dis_ref[...] * jnp.dot(x_ref[...], w_ref[...],
                                        preferred_element_type=jnp.float32)


def _epilogue_body(agg_ref, x_ref, b_ref, dis_ref, y_ref, out_ref):
    s = agg_ref[0] + agg_ref[1] + y_ref[...]
    out_ref[...] = jnp.maximum(dis_ref[...] * s + b_ref[...], 0.0) + x_ref[...]


def kernel(x, edge_index, W, b):
    ei32 = edge_index.astype(jnp.int32)
    pad = jnp.stack([
        jnp.full((NPAD,), N_NODES, jnp.int32),
        jnp.arange(NPAD, dtype=jnp.int32) % N_NODES,
    ])
    ei = jnp.concatenate([ei32, pad], axis=1).reshape(2, NCHUNKP, CH)
    zeros128 = jnp.zeros((N_NODES, D), jnp.float32)

    deg_hist = pl.kernel(
        _deg_hist_body,
        out_type=jax.ShapeDtypeStruct((NW * N_NODES,), jnp.float32),
        mesh=_mesh,
        compiler_params=_sc_params,
        scratch_types=[
            pltpu.VMEM((NCPT_DEG, CH), jnp.int32),
            pltpu.VMEM((N_NODES,), jnp.float32),
        ],
    )
    deg_parts = deg_hist(ei).reshape(NW, N_NODES)

    dis = pl.pallas_call(
        _dis_body,
        in_specs=[pl.BlockSpec((NW, N_NODES), lambda: (0, 0))],
        out_specs=pl.BlockSpec((N_NODES, 1), lambda: (0, 0)),
        out_shape=jax.ShapeDtypeStruct((N_NODES, 1), jnp.float32),
    )(deg_parts)

    R = 1000
    y = pl.pallas_call(
        _linear_body,
        grid=(N_NODES // R,),
        in_specs=[
            pl.BlockSpec((R, D), lambda i: (i, 0)),
            pl.BlockSpec((D, D), lambda i: (0, 0)),
            pl.BlockSpec((R, 1), lambda i: (i, 0)),
        ],
        out_specs=pl.BlockSpec((R, D), lambda i: (i, 0)),
        out_shape=jax.ShapeDtypeStruct((N_NODES, D), jnp.float32),
    )(x, W, dis)

    agg_call = pl.kernel(
        _agg_body,
        out_type=jax.ShapeDtypeStruct((NC, N_NODES, D), jnp.float32),
        mesh=_mesh,
        scratch_types=[
            pltpu.VMEM((GRP, CH), jnp.int32),
            pltpu.VMEM((GRP, CH), jnp.int32),
            pltpu.VMEM((CH, D), jnp.float32),
            pltpu.VMEM((CH, D), jnp.float32),
            pltpu.SemaphoreType.DMA,
            pltpu.SemaphoreType.DMA,
            pltpu.SemaphoreType.DMA,
            pltpu.SemaphoreType.DMA,
            pltpu.VMEM_SHARED((N_NODES, D), jnp.float32),
        ],
    )
    y_pad = jnp.concatenate([y, jnp.zeros((8, D), jnp.float32)], axis=0)
    agg = agg_call(y_pad, ei, zeros128)

    out = pl.pallas_call(
        _epilogue_body,
        grid=(N_NODES // R,),
        in_specs=[
            pl.BlockSpec((NC, R, D), lambda i: (0, i, 0)),
            pl.BlockSpec((R, D), lambda i: (i, 0)),
            pl.BlockSpec((1, D), lambda i: (0, 0)),
            pl.BlockSpec((R, 1), lambda i: (i, 0)),
            pl.BlockSpec((R, D), lambda i: (i, 0)),
        ],
        out_specs=pl.BlockSpec((R, D), lambda i: (i, 0)),
        out_shape=jax.ShapeDtypeStruct((N_NODES, D), jnp.float32),
    )(agg, x, b.reshape(1, D), dis, y)

    return (out, edge_index)


# speedup vs baseline: 1.1372x; 1.1372x over previous
import dataclasses

import jax
import jax.numpy as jnp
from jax import lax
from jax.experimental import pallas as pl
from jax.experimental.pallas import tpu as pltpu
from jax.experimental.pallas import tpu_sc as plsc

N_NODES = 10000
D = 128
N_EDGES = 320000

NC = 2
NS = 16
NW = NC * NS
CH = 128
NCHUNK = N_EDGES // CH
NCHUNKP = 2560
NPAD = NCHUNKP * CH - N_EDGES

SPAN = 624
TAIL_BASE = NS * SPAN
TAIL = N_NODES - TAIL_BASE

NCPT_DEG = NCHUNKP // NW

NCPT0 = 128
NCPT1 = 32
CORE1_BASE = NS * NCPT0
GRP = 16
NGRP0 = NCPT0 // GRP
NGRP1 = NCPT1 // GRP
SLOTS = 2
GITER = GRP // SLOTS

_mesh = plsc.VectorSubcoreMesh(core_axis_name="c", subcore_axis_name="s")

_sc_params = pltpu.CompilerParams()
if "needs_layout_passes" in pltpu.CompilerParams.__dataclass_fields__:
    _sc_params = dataclasses.replace(_sc_params, needs_layout_passes=False)


def _span_copy(sid, src, dst):
    base = sid * SPAN
    pltpu.sync_copy(src.at[pl.ds(base, SPAN)], dst.at[pl.ds(base, SPAN)])

    @pl.when(sid == NS - 1)
    def _():
        pltpu.sync_copy(src.at[pl.ds(TAIL_BASE, TAIL)],
                        dst.at[pl.ds(TAIL_BASE, TAIL)])


def _deg_hist_body(ei_hbm, out_hbm, idx_v, deg_v):
    cid = lax.axis_index("c")
    sid = lax.axis_index("s")
    wid = sid * NC + cid
    start = wid * NCPT_DEG

    @pl.loop(0, N_NODES // 16)
    def _(r):
        deg_v[pl.ds(r * 16, 16)] = jnp.zeros((16,), jnp.float32)

    pltpu.sync_copy(ei_hbm.at[1, pl.ds(start, NCPT_DEG)], idx_v)

    ones = jnp.ones((16,), jnp.float32)
    nloc = jnp.clip(NCHUNK - start, 0, NCPT_DEG)

    @pl.loop(0, nloc)
    def _(c):
        for j in range(CH // 16):
            idx16 = idx_v[c, pl.ds(j * 16, 16)]
            plsc.addupdate_scatter(deg_v, [idx16], ones)

    pltpu.sync_copy(deg_v, out_hbm.at[pl.ds(wid * N_NODES, N_NODES)])


def _agg_body(y_hbm, ei_hbm, zeros_hbm, out_hbm,
              rowi_v, coli_v, buf0, buf1, g0, g1, s0, s1, acc_sh):
    cid = lax.axis_index("c")
    sid = lax.axis_index("s")
    bufs = (buf0, buf1)
    gsems = (g0, g1)
    ssems = (s0, s1)

    _span_copy(sid, zeros_hbm, acc_sh)
    plsc.subcore_barrier()

    def load_idx(gs):
        pltpu.sync_copy(ei_hbm.at[0, pl.ds(gs, GRP)], rowi_v)
        pltpu.sync_copy(ei_hbm.at[1, pl.ds(gs, GRP)], coli_v)

    def g_start(b, j):
        pltpu.make_async_copy(y_hbm.at[rowi_v.at[j]], bufs[b],
                              gsems[b]).start()

    def g_wait(b):
        pltpu.make_async_copy(y_hbm.at[rowi_v.at[0]], bufs[b],
                              gsems[b]).wait()

    def s_start(b, j):
        pltpu.make_async_copy(bufs[b], acc_sh.at[coli_v.at[j]],
                              ssems[b]).start(add=True)

    def s_wait(b):
        pltpu.make_async_copy(bufs[b], acc_sh.at[coli_v.at[0]],
                              ssems[b]).wait()

    start = jnp.where(cid == 0, sid * NCPT0, CORE1_BASE + sid * NCPT1)
    ngroups = jnp.where(cid == 0, NGRP0, NGRP1)

    @pl.loop(0, ngroups)
    def _(g):
        load_idx(start + g * GRP)
        for b in range(SLOTS):
            g_start(b, b)

        @pl.loop(0, GITER)
        def _(i):
            base = i * SLOTS
            for b in range(SLOTS):
                g_wait(b)
                s_start(b, base + b)
            for b in range(SLOTS):
                s_wait(b)
                nxt = base + SLOTS + b

                @pl.when(nxt < GRP)
                def _():
                    g_start(b, nxt)

    plsc.subcore_barrier()
    _span_copy(sid, acc_sh, out_hbm.at[cid])


def _dis_body(parts_ref, dis_ref):
    deg = jnp.sum(parts_ref[...], axis=0, keepdims=True) + 1.0
    dis_ref[...] = jnp.transpose(lax.rsqrt(deg), (1, 0))


def _linear_body(x_ref, w_ref, dis_ref, y_ref):
    y_ref[...] = dis_ref[...] * jnp.dot(x_ref[...], w_ref[...],
                                        preferred_element_type=jnp.float32)


def _epilogue_body(agg_ref, x_ref, b_ref, dis_ref, y_ref, out_ref):
    s = agg_ref[0] + agg_ref[1] + y_ref[...]
    out_ref[...] = jnp.maximum(dis_ref[...] * s + b_ref[...], 0.0) + x_ref[...]


def kernel(x, edge_index, W, b):
    ei32 = edge_index.astype(jnp.int32)
    pad = jnp.stack([
        jnp.full((NPAD,), N_NODES, jnp.int32),
        jnp.arange(NPAD, dtype=jnp.int32) % N_NODES,
    ])
    ei = jnp.concatenate([ei32, pad], axis=1).reshape(2, NCHUNKP, CH)
    zeros128 = jnp.zeros((N_NODES, D), jnp.float32)

    deg_hist = pl.kernel(
        _deg_hist_body,
        out_type=jax.ShapeDtypeStruct((NW * N_NODES,), jnp.float32),
        mesh=_mesh,
        compiler_params=_sc_params,
        scratch_types=[
            pltpu.VMEM((NCPT_DEG, CH), jnp.int32),
            pltpu.VMEM((N_NODES,), jnp.float32),
        ],
    )
    deg_parts = deg_hist(ei).reshape(NW, N_NODES)

    dis = pl.pallas_call(
        _dis_body,
        in_specs=[pl.BlockSpec((NW, N_NODES), lambda: (0, 0))],
        out_specs=pl.BlockSpec((N_NODES, 1), lambda: (0, 0)),
        out_shape=jax.ShapeDtypeStruct((N_NODES, 1), jnp.float32),
    )(deg_parts)

    R = 1000
    y = pl.pallas_call(
        _linear_body,
        grid=(N_NODES // R,),
        in_specs=[
            pl.BlockSpec((R, D), lambda i: (i, 0)),
            pl.BlockSpec((D, D), lambda i: (0, 0)),
            pl.BlockSpec((R, 1), lambda i: (i, 0)),
        ],
        out_specs=pl.BlockSpec((R, D), lambda i: (i, 0)),
        out_shape=jax.ShapeDtypeStruct((N_NODES, D), jnp.float32),
    )(x, W, dis)

    agg_call = pl.kernel(
        _agg_body,
        out_type=jax.ShapeDtypeStruct((NC, N_NODES, D), jnp.float32),
        mesh=_mesh,
        scratch_types=[
            pltpu.VMEM((GRP, CH), jnp.int32),
            pltpu.VMEM((GRP, CH), jnp.int32),
            pltpu.VMEM((CH, D), jnp.float32),
            pltpu.VMEM((CH, D), jnp.float32),
            pltpu.SemaphoreType.DMA,
            pltpu.SemaphoreType.DMA,
            pltpu.SemaphoreType.DMA,
            pltpu.SemaphoreType.DMA,
            pltpu.VMEM_SHARED((N_NODES, D), jnp.float32),
        ],
    )
    y_pad = jnp.concatenate([y, jnp.zeros((8, D), jnp.float32)], axis=0)
    agg = agg_call(y_pad, ei, zeros128)

    out = pl.pallas_call(
        _epilogue_body,
        grid=(N_NODES // R,),
        in_specs=[
            pl.BlockSpec((NC, R, D), lambda i: (0, i, 0)),
            pl.BlockSpec((R, D), lambda i: (i, 0)),
            pl.BlockSpec((1, D), lambda i: (0, 0)),
            pl.BlockSpec((R, 1), lambda i: (i, 0)),
            pl.BlockSpec((R, D), lambda i: (i, 0)),
        ],
        out_specs=pl.BlockSpec((R, D), lambda i: (i, 0)),
        out_shape=jax.ShapeDtypeStruct((N_NODES, D), jnp.float32),
    )(agg, x, b.reshape(1, D), dis, y)

    return (out, edge_index)


# speedup vs baseline: 2.7612x; 2.4280x over previous
import dataclasses

import jax
import jax.numpy as jnp
from jax import lax
from jax.experimental import pallas as pl
from jax.experimental.pallas import tpu as pltpu
from jax.experimental.pallas import tpu_sc as plsc

N_NODES = 10000
D = 128
N_EDGES = 320000

NC = 2
NS = 16
NW = NC * NS
CH = 128
NCHUNK = N_EDGES // CH
NCHUNKP = 2560
NPAD = NCHUNKP * CH - N_EDGES

SPAN = 624
TAIL_BASE = NS * SPAN
TAIL = N_NODES - TAIL_BASE

NCPT_DEG = NCHUNKP // NW

NCPT0 = 80
NCPT1 = 80
CORE1_BASE = NS * NCPT0
ZROWS = 128
GRP = 16
NGRP0 = NCPT0 // GRP
NGRP1 = NCPT1 // GRP
SLOTS = 2
GITER = GRP // SLOTS

_mesh = plsc.VectorSubcoreMesh(core_axis_name="c", subcore_axis_name="s")

_sc_params = pltpu.CompilerParams()
if "needs_layout_passes" in pltpu.CompilerParams.__dataclass_fields__:
    _sc_params = dataclasses.replace(_sc_params, needs_layout_passes=False)


def _span_copy(sid, src, dst):
    base = sid * SPAN
    pltpu.sync_copy(src.at[pl.ds(base, SPAN)], dst.at[pl.ds(base, SPAN)])

    @pl.when(sid == NS - 1)
    def _():
        pltpu.sync_copy(src.at[pl.ds(TAIL_BASE, TAIL)],
                        dst.at[pl.ds(TAIL_BASE, TAIL)])


def _deg_hist_body(ei_hbm, out_hbm, idx_v, deg_v):
    cid = lax.axis_index("c")
    sid = lax.axis_index("s")
    wid = sid * NC + cid
    start = wid * NCPT_DEG

    @pl.loop(0, N_NODES // 16)
    def _(r):
        deg_v[pl.ds(r * 16, 16)] = jnp.zeros((16,), jnp.float32)

    pltpu.sync_copy(ei_hbm.at[1, pl.ds(start, NCPT_DEG)], idx_v)

    ones = jnp.ones((16,), jnp.float32)
    nloc = jnp.clip(NCHUNK - start, 0, NCPT_DEG)

    @pl.loop(0, nloc)
    def _(c):
        for j in range(CH // 16):
            idx16 = idx_v[c, pl.ds(j * 16, 16)]
            plsc.addupdate_scatter(deg_v, [idx16], ones)

    pltpu.sync_copy(deg_v, out_hbm.at[pl.ds(wid * N_NODES, N_NODES)])


def _agg_body(y_hbm, ei_hbm, zeros_hbm, out_hbm,
              rowi_v, coli_v, buf0, buf1, g0, g1, s0, s1, acc_sh):
    cid = lax.axis_index("c")
    sid = lax.axis_index("s")
    bufs = (buf0, buf1)
    gsems = (g0, g1)
    ssems = (s0, s1)

    _span_copy(sid, zeros_hbm, acc_sh)
    plsc.subcore_barrier()

    def load_idx(gs):
        pltpu.sync_copy(ei_hbm.at[0, pl.ds(gs, GRP)], rowi_v)
        pltpu.sync_copy(ei_hbm.at[1, pl.ds(gs, GRP)], coli_v)

    def g_start(b, j):
        pltpu.make_async_copy(y_hbm.at[rowi_v.at[j]], bufs[b],
                              gsems[b]).start()

    def g_wait(b):
        pltpu.make_async_copy(y_hbm.at[rowi_v.at[0]], bufs[b],
                              gsems[b]).wait()

    def s_start(b, j):
        pltpu.make_async_copy(bufs[b], acc_sh.at[coli_v.at[j]],
                              ssems[b]).start(add=True)

    def s_wait(b):
        pltpu.make_async_copy(bufs[b], acc_sh.at[coli_v.at[0]],
                              ssems[b]).wait()

    start = jnp.where(cid == 0, sid * NCPT0, CORE1_BASE + sid * NCPT1)
    ngroups = jnp.where(cid == 0, NGRP0, NGRP1)

    @pl.loop(0, ngroups)
    def _(g):
        load_idx(start + g * GRP)
        for b in range(SLOTS):
            g_start(b, b)

        @pl.loop(0, GITER)
        def _(i):
            base = i * SLOTS
            for b in range(SLOTS):
                g_wait(b)
                s_start(b, base + b)
            for b in range(SLOTS):
                s_wait(b)
                nxt = base + SLOTS + b

                @pl.when(nxt < GRP)
                def _():
                    g_start(b, nxt)

    plsc.subcore_barrier()
    _span_copy(sid, acc_sh, out_hbm.at[cid])


def _dis_body(parts_ref, dis_ref):
    deg = jnp.sum(parts_ref[...], axis=0, keepdims=True) + 1.0
    dis_ref[...] = jnp.transpose(lax.rsqrt(deg), (1, 0))


def _linear_body(x_ref, w_ref, dis_ref, y_ref):
    y_ref[...] = dis_ref[...] * jnp.dot(x_ref[...], w_ref[...],
                                        preferred_element_type=jnp.float32)


def _epilogue_body(agg_ref, x_ref, b_ref, dis_ref, y_ref, out_ref):
    s = agg_ref[0] + agg_ref[1] + y_ref[...]
    out_ref[...] = jnp.maximum(dis_ref[...] * s + b_ref[...], 0.0) + x_ref[...]


def kernel(x, edge_index, W, b):
    ei32 = edge_index.astype(jnp.int32)
    ar = jnp.arange(NPAD, dtype=jnp.int32)
    pad = jnp.stack([
        N_NODES + ar % ZROWS,
        ar % N_NODES,
    ])
    ei = jnp.concatenate([ei32, pad], axis=1).reshape(2, NCHUNKP, CH)
    zeros128 = jnp.zeros((N_NODES, D), jnp.float32)

    deg_hist = pl.kernel(
        _deg_hist_body,
        out_type=jax.ShapeDtypeStruct((NW * N_NODES,), jnp.float32),
        mesh=_mesh,
        compiler_params=_sc_params,
        scratch_types=[
            pltpu.VMEM((NCPT_DEG, CH), jnp.int32),
            pltpu.VMEM((N_NODES,), jnp.float32),
        ],
    )
    deg_parts = deg_hist(ei).reshape(NW, N_NODES)

    dis = pl.pallas_call(
        _dis_body,
        in_specs=[pl.BlockSpec((NW, N_NODES), lambda: (0, 0))],
        out_specs=pl.BlockSpec((N_NODES, 1), lambda: (0, 0)),
        out_shape=jax.ShapeDtypeStruct((N_NODES, 1), jnp.float32),
    )(deg_parts)

    R = 1000
    y = pl.pallas_call(
        _linear_body,
        grid=(N_NODES // R,),
        in_specs=[
            pl.BlockSpec((R, D), lambda i: (i, 0)),
            pl.BlockSpec((D, D), lambda i: (0, 0)),
            pl.BlockSpec((R, 1), lambda i: (i, 0)),
        ],
        out_specs=pl.BlockSpec((R, D), lambda i: (i, 0)),
        out_shape=jax.ShapeDtypeStruct((N_NODES, D), jnp.float32),
    )(x, W, dis)

    agg_call = pl.kernel(
        _agg_body,
        out_type=jax.ShapeDtypeStruct((NC, N_NODES, D), jnp.float32),
        mesh=_mesh,
        scratch_types=[
            pltpu.VMEM((GRP, CH), jnp.int32),
            pltpu.VMEM((GRP, CH), jnp.int32),
            pltpu.VMEM((CH, D), jnp.float32),
            pltpu.VMEM((CH, D), jnp.float32),
            pltpu.SemaphoreType.DMA,
            pltpu.SemaphoreType.DMA,
            pltpu.SemaphoreType.DMA,
            pltpu.SemaphoreType.DMA,
            pltpu.VMEM_SHARED((N_NODES, D), jnp.float32),
        ],
    )
    y_pad = jnp.concatenate([y, jnp.zeros((ZROWS, D), jnp.float32)], axis=0)
    agg = agg_call(y_pad, ei, zeros128)

    out = pl.pallas_call(
        _epilogue_body,
        grid=(N_NODES // R,),
        in_specs=[
            pl.BlockSpec((NC, R, D), lambda i: (0, i, 0)),
            pl.BlockSpec((R, D), lambda i: (i, 0)),
            pl.BlockSpec((1, D), lambda i: (0, 0)),
            pl.BlockSpec((R, 1), lambda i: (i, 0)),
            pl.BlockSpec((R, D), lambda i: (i, 0)),
        ],
        out_specs=pl.BlockSpec((R, D), lambda i: (i, 0)),
        out_shape=jax.ShapeDtypeStruct((N_NODES, D), jnp.float32),
    )(agg, x, b.reshape(1, D), dis, y)

    return (out, edge_index)


# speedup vs baseline: 2.9295x; 1.0610x over previous
import dataclasses

import jax
import jax.numpy as jnp
from jax import lax
from jax.experimental import pallas as pl
from jax.experimental.pallas import tpu as pltpu
from jax.experimental.pallas import tpu_sc as plsc

N_NODES = 10000
D = 128
N_EDGES = 320000

NC = 2
NS = 16
NW = NC * NS
CH = 64
NCHUNK = N_EDGES // CH
NCHUNKP = 5120
NPAD = NCHUNKP * CH - N_EDGES

SPAN = 624
TAIL_BASE = NS * SPAN
TAIL = N_NODES - TAIL_BASE

NCPT_DEG = NCHUNKP // NW

NCPT0 = 160
NCPT1 = 160
CORE1_BASE = NS * NCPT0
ZROWS = 128
GRP = 16
NGRP0 = NCPT0 // GRP
NGRP1 = NCPT1 // GRP
SLOTS = 4
GITER = GRP // SLOTS

_mesh = plsc.VectorSubcoreMesh(core_axis_name="c", subcore_axis_name="s")

_sc_params = pltpu.CompilerParams()
if "needs_layout_passes" in pltpu.CompilerParams.__dataclass_fields__:
    _sc_params = dataclasses.replace(_sc_params, needs_layout_passes=False)


def _span_copy(sid, src, dst):
    base = sid * SPAN
    pltpu.sync_copy(src.at[pl.ds(base, SPAN)], dst.at[pl.ds(base, SPAN)])

    @pl.when(sid == NS - 1)
    def _():
        pltpu.sync_copy(src.at[pl.ds(TAIL_BASE, TAIL)],
                        dst.at[pl.ds(TAIL_BASE, TAIL)])


def _deg_hist_body(ei_hbm, out_hbm, idx_v, deg_v):
    cid = lax.axis_index("c")
    sid = lax.axis_index("s")
    wid = sid * NC + cid
    start = wid * NCPT_DEG

    @pl.loop(0, N_NODES // 16)
    def _(r):
        deg_v[pl.ds(r * 16, 16)] = jnp.zeros((16,), jnp.float32)

    pltpu.sync_copy(ei_hbm.at[1, pl.ds(start, NCPT_DEG)], idx_v)

    ones = jnp.ones((16,), jnp.float32)
    nloc = jnp.clip(NCHUNK - start, 0, NCPT_DEG)

    @pl.loop(0, nloc)
    def _(c):
        for j in range(CH // 16):
            idx16 = idx_v[c, pl.ds(j * 16, 16)]
            plsc.addupdate_scatter(deg_v, [idx16], ones)

    pltpu.sync_copy(deg_v, out_hbm.at[pl.ds(wid * N_NODES, N_NODES)])


def _agg_body(y_hbm, ei_hbm, zeros_hbm, out_hbm,
              rowi_v, coli_v, buf0, buf1, buf2, buf3,
              g0, g1, g2, g3, s0, s1, s2, s3, acc_sh):
    cid = lax.axis_index("c")
    sid = lax.axis_index("s")
    bufs = (buf0, buf1, buf2, buf3)
    gsems = (g0, g1, g2, g3)
    ssems = (s0, s1, s2, s3)

    _span_copy(sid, zeros_hbm, acc_sh)
    plsc.subcore_barrier()

    def load_idx(gs):
        pltpu.sync_copy(ei_hbm.at[0, pl.ds(gs, GRP)], rowi_v)
        pltpu.sync_copy(ei_hbm.at[1, pl.ds(gs, GRP)], coli_v)

    def g_start(b, j):
        pltpu.make_async_copy(y_hbm.at[rowi_v.at[j]], bufs[b],
                              gsems[b]).start()

    def g_wait(b):
        pltpu.make_async_copy(y_hbm.at[rowi_v.at[0]], bufs[b],
                              gsems[b]).wait()

    def s_start(b, j):
        pltpu.make_async_copy(bufs[b], acc_sh.at[coli_v.at[j]],
                              ssems[b]).start(add=True)

    def s_wait(b):
        pltpu.make_async_copy(bufs[b], acc_sh.at[coli_v.at[0]],
                              ssems[b]).wait()

    start = jnp.where(cid == 0, sid * NCPT0, CORE1_BASE + sid * NCPT1)
    ngroups = jnp.where(cid == 0, NGRP0, NGRP1)

    @pl.loop(0, ngroups)
    def _(g):
        load_idx(start + g * GRP)
        for b in range(SLOTS):
            g_start(b, b)

        @pl.loop(0, GITER)
        def _(i):
            base = i * SLOTS
            for b in range(SLOTS):
                g_wait(b)
                s_start(b, base + b)
            for b in range(SLOTS):
                s_wait(b)
                nxt = base + SLOTS + b

                @pl.when(nxt < GRP)
                def _():
                    g_start(b, nxt)

    plsc.subcore_barrier()
    _span_copy(sid, acc_sh, out_hbm.at[cid])


def _dis_body(parts_ref, dis_ref):
    deg = jnp.sum(parts_ref[...], axis=0, keepdims=True) + 1.0
    dis_ref[...] = jnp.transpose(lax.rsqrt(deg), (1, 0))


def _linear_body(x_ref, w_ref, dis_ref, y_ref):
    y_ref[...] = dis_ref[...] * jnp.dot(x_ref[...], w_ref[...],
                                        preferred_element_type=jnp.float32)


def _epilogue_body(agg_ref, x_ref, b_ref, dis_ref, y_ref, out_ref):
    s = agg_ref[0] + agg_ref[1] + y_ref[...]
    out_ref[...] = jnp.maximum(dis_ref[...] * s + b_ref[...], 0.0) + x_ref[...]


def kernel(x, edge_index, W, b):
    ei32 = edge_index.astype(jnp.int32)
    ar = jnp.arange(NPAD, dtype=jnp.int32)
    pad = jnp.stack([
        N_NODES + ar % ZROWS,
        ar % N_NODES,
    ])
    ei = jnp.concatenate([ei32, pad], axis=1).reshape(2, NCHUNKP, CH)
    zeros128 = jnp.zeros((N_NODES, D), jnp.float32)

    deg_hist = pl.kernel(
        _deg_hist_body,
        out_type=jax.ShapeDtypeStruct((NW * N_NODES,), jnp.float32),
        mesh=_mesh,
        compiler_params=_sc_params,
        scratch_types=[
            pltpu.VMEM((NCPT_DEG, CH), jnp.int32),
            pltpu.VMEM((N_NODES,), jnp.float32),
        ],
    )
    deg_parts = deg_hist(ei).reshape(NW, N_NODES)

    dis = pl.pallas_call(
        _dis_body,
        in_specs=[pl.BlockSpec((NW, N_NODES), lambda: (0, 0))],
        out_specs=pl.BlockSpec((N_NODES, 1), lambda: (0, 0)),
        out_shape=jax.ShapeDtypeStruct((N_NODES, 1), jnp.float32),
    )(deg_parts)

    R = 1000
    y = pl.pallas_call(
        _linear_body,
        grid=(N_NODES // R,),
        in_specs=[
            pl.BlockSpec((R, D), lambda i: (i, 0)),
            pl.BlockSpec((D, D), lambda i: (0, 0)),
            pl.BlockSpec((R, 1), lambda i: (i, 0)),
        ],
        out_specs=pl.BlockSpec((R, D), lambda i: (i, 0)),
        out_shape=jax.ShapeDtypeStruct((N_NODES, D), jnp.float32),
    )(x, W, dis)

    agg_call = pl.kernel(
        _agg_body,
        out_type=jax.ShapeDtypeStruct((NC, N_NODES, D), jnp.float32),
        mesh=_mesh,
        scratch_types=[
            pltpu.VMEM((GRP, CH), jnp.int32),
            pltpu.VMEM((GRP, CH), jnp.int32),
            pltpu.VMEM((CH, D), jnp.float32),
            pltpu.VMEM((CH, D), jnp.float32),
            pltpu.VMEM((CH, D), jnp.float32),
            pltpu.VMEM((CH, D), jnp.float32),
            pltpu.SemaphoreType.DMA,
            pltpu.SemaphoreType.DMA,
            pltpu.SemaphoreType.DMA,
            pltpu.SemaphoreType.DMA,
            pltpu.SemaphoreType.DMA,
            pltpu.SemaphoreType.DMA,
            pltpu.SemaphoreType.DMA,
            pltpu.SemaphoreType.DMA,
            pltpu.VMEM_SHARED((N_NODES, D), jnp.float32),
        ],
    )
    y_pad = jnp.concatenate([y, jnp.zeros((ZROWS, D), jnp.float32)], axis=0)
    agg = agg_call(y_pad, ei, zeros128)

    out = pl.pallas_call(
        _epilogue_body,
        grid=(N_NODES // R,),
        in_specs=[
            pl.BlockSpec((NC, R, D), lambda i: (0, i, 0)),
            pl.BlockSpec((R, D), lambda i: (i, 0)),
            pl.BlockSpec((1, D), lambda i: (0, 0)),
            pl.BlockSpec((R, 1), lambda i: (i, 0)),
            pl.BlockSpec((R, D), lambda i: (i, 0)),
        ],
        out_specs=pl.BlockSpec((R, D), lambda i: (i, 0)),
        out_shape=jax.ShapeDtypeStruct((N_NODES, D), jnp.float32),
    )(agg, x, b.reshape(1, D), dis, y)

    return (out, edge_index)
